# Initial kernel scaffold; baseline (speedup 1.0000x reference)
import jax
import jax.numpy as jnp
from jax.experimental import pallas as pl


def kernel(edge_index, feats, edge_type, edge_norm, V1, comp1, bias1, loop1, V2, comp2, bias2, loop2, V3, comp3, bias3, loop3):
    raise NotImplementedError("write your pallas kernel here")


# speedup vs baseline: 2.7044x; 2.7044x over previous
import functools

import jax
import jax.numpy as jnp
from jax import lax
from jax.experimental import pallas as pl
from jax.experimental.pallas import tpu as pltpu
from jax.experimental.pallas import tpu_sc as plsc

_NC = 2
_NS = 16
_LANES = 16
_NW = _NC * _NS

_CHUNK = 80


def _sc_edge_pass(table, edata, norm_r, n_nodes, d):
    nchunk = edata.shape[1]
    epw = nchunk * _CHUNK
    rps = (n_nodes // _NS) // 8 * 8
    tail = n_nodes - _NS * rps
    zr = 24
    nz = rps // zr
    ngrp = _CHUNK // _LANES
    ncol = d // _LANES

    mesh = plsc.VectorSubcoreMesh(core_axis_name="c", subcore_axis_name="s")

    @functools.partial(
        pl.kernel,
        out_type=jax.ShapeDtypeStruct((_NC, n_nodes, d), jnp.float32),
        mesh=mesh,
        scratch_types=[
            pltpu.VMEM_SHARED((n_nodes, d), jnp.float32),
            pltpu.VMEM((3, _CHUNK), jnp.int32),
            pltpu.VMEM((1, _CHUNK), jnp.float32),
            pltpu.VMEM((_CHUNK,), jnp.int32),
            pltpu.VMEM((_CHUNK, d), jnp.float32),
            pltpu.VMEM((zr, d), jnp.float32),
            pltpu.SemaphoreType.DMA,
        ],
    )
    def kfn(edata_h, norm_h, table_h, out_h, acc, ebuf, nbuf, idx_v, rows,
            zbuf, sem):
        cid = lax.axis_index("c")
        sid = lax.axis_index("s")
        wid = sid * _NC + cid

        zeros16 = jnp.zeros((_LANES,), jnp.float32)

        def zb_body(i, _):
            for c in range(ncol):
                zbuf[i, pl.ds(c * _LANES, _LANES)] = zeros16
            return 0
        lax.fori_loop(0, zr, zb_body, 0)
        row0 = sid * rps

        def zc_body(t, _):
            pltpu.sync_copy(zbuf, acc.at[pl.ds(row0 + t * zr, zr)])
            return 0
        lax.fori_loop(0, nz, zc_body, 0)

        @pl.when(sid == _NS - 1)
        def _():
            pltpu.sync_copy(zbuf.at[pl.ds(0, tail)],
                            acc.at[pl.ds(_NS * rps, tail)])
        plsc.subcore_barrier()

        def chunk_body(k, _):
            pltpu.sync_copy(edata_h.at[wid, k], ebuf)
            pltpu.sync_copy(norm_h.at[wid, k], nbuf)

            def gidx(g, _):
                s16 = ebuf[0, pl.ds(g * _LANES, _LANES)]
                e16 = ebuf[1, pl.ds(g * _LANES, _LANES)]
                idx_v[pl.ds(g * _LANES, _LANES)] = e16 * n_nodes + s16
                return 0
            lax.fori_loop(0, ngrp, gidx, 0)

            pltpu.async_copy(table_h.at[idx_v], rows, sem).wait()

            def scale(g, _):
                n16 = nbuf[0, pl.ds(g * _LANES, _LANES)]
                row0g = g * _LANES
                for e in range(_LANES):
                    nb = jnp.full((_LANES,), n16[e])
                    for c in range(ncol):
                        v = rows[row0g + e, pl.ds(c * _LANES, _LANES)]
                        rows[row0g + e, pl.ds(c * _LANES, _LANES)] = v * nb
                return 0
            lax.fori_loop(0, ngrp, scale, 0)

            pltpu.sync_copy(rows, acc.at[ebuf.at[2]], add=True)
            return 0
        lax.fori_loop(0, nchunk, chunk_body, 0)

        plsc.subcore_barrier()

        pltpu.sync_copy(acc.at[pl.ds(row0, rps)],
                        out_h.at[cid, pl.ds(row0, rps)])

        @pl.when(sid == _NS - 1)
        def _():
            pltpu.sync_copy(acc.at[pl.ds(_NS * rps, tail)],
                            out_h.at[cid, pl.ds(_NS * rps, tail)])

    return kfn(edata, norm_r, table)


def _basis_combine(comp, vflat):
    r1, b1 = comp.shape
    k = vflat.shape[1]
    nb = 25 if k > (1 << 16) else 1
    cb = k // nb

    def body(c_ref, v_ref, o_ref):
        o_ref[...] = jnp.dot(c_ref[...], v_ref[...],
                             preferred_element_type=jnp.float32)

    return pl.pallas_call(
        body,
        grid=(nb,),
        in_specs=[pl.BlockSpec((r1, b1), lambda i: (0, 0)),
                  pl.BlockSpec((b1, cb), lambda i: (0, i))],
        out_specs=pl.BlockSpec((r1, cb), lambda i: (0, i)),
        out_shape=jax.ShapeDtypeStruct((r1, k), jnp.float32),
    )(comp, vflat)


def _relu_combine(p, self_term, bias, nblk):
    n, d = self_term.shape
    nb = n // nblk

    def body(p_ref, s_ref, b_ref, o_ref):
        o_ref[...] = jnp.maximum(
            p_ref[0] + p_ref[1] + s_ref[...] + b_ref[...], 0.0)

    return pl.pallas_call(
        body,
        grid=(nb,),
        in_specs=[pl.BlockSpec((2, nblk, d), lambda i: (0, i, 0)),
                  pl.BlockSpec((nblk, d), lambda i: (i, 0)),
                  pl.BlockSpec((1, d), lambda i: (0, 0))],
        out_specs=pl.BlockSpec((nblk, d), lambda i: (i, 0)),
        out_shape=jax.ShapeDtypeStruct((n, d), jnp.float32),
    )(p, self_term, bias.reshape(1, d))


def _project_all(h, w, nblk):
    n, hdim = h.shape
    r1, _, do = w.shape
    nb = n // nblk

    def body(h_ref, w_ref, o_ref):
        o_ref[0] = jnp.dot(h_ref[...], w_ref[0],
                           preferred_element_type=jnp.float32)

    return pl.pallas_call(
        body,
        grid=(nb, r1),
        in_specs=[pl.BlockSpec((nblk, hdim), lambda i, r: (i, 0)),
                  pl.BlockSpec((1, hdim, do), lambda i, r: (r, 0, 0))],
        out_specs=pl.BlockSpec((1, nblk, do), lambda i, r: (r, i, 0)),
        out_shape=jax.ShapeDtypeStruct((r1, n, do), jnp.float32),
    )(h, w)


def _masked_softmax(p, self_term, bias, nout, nblk):
    n, d = self_term.shape
    nb = n // nblk

    def body(p_ref, s_ref, b_ref, o_ref):
        x = p_ref[0] + p_ref[1] + s_ref[...] + b_ref[...]
        col = lax.broadcasted_iota(jnp.int32, (nblk, d), 1)
        x = jnp.where(col >= nout, -1e30, x)
        m = jnp.max(x, axis=1, keepdims=True)
        e = jnp.exp(x - m)
        o_ref[...] = e / jnp.sum(e, axis=1, keepdims=True)

    return pl.pallas_call(
        body,
        grid=(nb,),
        in_specs=[pl.BlockSpec((2, nblk, d), lambda i: (0, i, 0)),
                  pl.BlockSpec((nblk, d), lambda i: (i, 0)),
                  pl.BlockSpec((1, d), lambda i: (0, 0))],
        out_specs=pl.BlockSpec((nblk, d), lambda i: (i, 0)),
        out_shape=jax.ShapeDtypeStruct((n, d), jnp.float32),
    )(p, self_term, bias.reshape(1, d))


def kernel(edge_index, feats, edge_type, edge_norm,
           V1, comp1, bias1, loop1,
           V2, comp2, bias2, loop2,
           V3, comp3, bias3, loop3):
    n = loop1.shape[0]
    e = edge_index.shape[1]
    b, _, h = V1.shape
    r = comp1.shape[0]
    o = V3.shape[2]
    dpad = 128

    epw = e // _NW
    nchunk = epw // _CHUNK

    src = edge_index[0].astype(jnp.int32)
    dst = edge_index[1].astype(jnp.int32)
    ety = edge_type.astype(jnp.int32)
    norm = edge_norm[:, 0]

    edata = jnp.stack([src, ety, dst], axis=0)
    edata = edata.reshape(3, _NW, nchunk, _CHUNK).transpose(1, 2, 0, 3)
    norm_r = norm.reshape(_NW, nchunk, 1, _CHUNK)

    t1 = _basis_combine(comp1, V1.reshape(b, n * h)).reshape(r * n, h)
    p1 = _sc_edge_pass(t1, edata, norm_r, n, h)
    h1 = _relu_combine(p1, loop1, bias1, 400)

    ce2 = jnp.zeros((r + 1, b + 1), jnp.float32)
    ce2 = ce2.at[:r, :b].set(comp2).at[r, b].set(1.0)
    vs2 = jnp.concatenate([V2, loop2[None]], axis=0)
    w2 = _basis_combine(ce2, vs2.reshape(b + 1, h * h)).reshape(r + 1, h, h)
    t2 = _project_all(h1, w2, 400)
    p2 = _sc_edge_pass(t2.reshape((r + 1) * n, h), edata, norm_r, n, h)
    h2 = _relu_combine(p2, t2[r], bias2, 400)

    ce3 = jnp.zeros((r + 1, b + 1), jnp.float32)
    ce3 = ce3.at[:r, :b].set(comp3).at[r, b].set(1.0)
    vs3 = jnp.concatenate([V3, loop3[None]], axis=0)
    vs3 = jnp.pad(vs3, ((0, 0), (0, 0), (0, dpad - o)))
    w3 = _basis_combine(ce3, vs3.reshape(b + 1, h * dpad)).reshape(
        r + 1, h, dpad)
    t3 = _project_all(h2, w3, 400)
    p3 = _sc_edge_pass(t3.reshape((r + 1) * n, dpad), edata, norm_r,
                       n, dpad)
    bias3p = jnp.pad(bias3, (0, dpad - o))
    sm = _masked_softmax(p3, t3[r], bias3p, o, 400)
    return sm[:, :o]
